# baseline (speedup 1.0000x reference)
import functools

import jax
import jax.numpy as jnp
from jax import lax
from jax.experimental import pallas as pl
from jax.experimental.pallas import tpu as pltpu
from jax.experimental.pallas import tpu_sc as plsc

N = 10000
E = 40000
NG = 512
NC, NS = 2, 16
NW = NC * NS
CH = 128
K = (E + NW * CH - 1) // (NW * CH)
E_PAD = NW * K * CH
N_PAD = 10240
STRIPE = N_PAD // NS
TN = 1000
DIMS = [(64, 32), (32, 64), (64, 64)]


def _sc_mesh():
    return plsc.VectorSubcoreMesh(core_axis_name="c", subcore_axis_name="s",
                                  num_cores=NC, num_subcores=NS)


def _sc_gather(table, idx3, d):
    @functools.partial(
        pl.kernel,
        mesh=_sc_mesh(),
        out_type=jax.ShapeDtypeStruct((E_PAD, d), jnp.float32),
        scratch_types=[
            pltpu.VMEM((K, CH), jnp.int32),
            pltpu.VMEM((CH, d), jnp.float32),
            pltpu.SemaphoreType.DMA,
        ],
        compiler_params=pltpu.CompilerParams(use_tc_tiling_on_sc=False),
    )
    def k(table_hbm, idx_hbm, out_hbm, idx_v, rows_v, sem):
        wid = lax.axis_index("s") * NC + lax.axis_index("c")
        pltpu.sync_copy(idx_hbm.at[wid], idx_v)
        for j in range(K):
            pltpu.async_copy(table_hbm.at[idx_v.at[j]], rows_v, sem).wait()
            pltpu.sync_copy(rows_v, out_hbm.at[pl.ds((wid * K + j) * CH, CH)])

    return k(table, idx3)


def _sc_scatter(msg, dst3, zeros, d):
    @functools.partial(
        pl.kernel,
        mesh=_sc_mesh(),
        out_type=jax.ShapeDtypeStruct((NC, N_PAD, d), jnp.float32),
        scratch_types=[
            pltpu.VMEM((K, CH), jnp.int32),
            pltpu.VMEM((CH, d), jnp.float32),
            pltpu.VMEM_SHARED((N_PAD, d), jnp.float32),
            pltpu.SemaphoreType.DMA,
        ],
        compiler_params=pltpu.CompilerParams(use_tc_tiling_on_sc=False),
    )
    def k(msg_hbm, dst_hbm, z_hbm, out_hbm, idx_v, row_v, acc_sh, sem):
        c = lax.axis_index("c")
        s = lax.axis_index("s")
        wid = s * NC + c
        pltpu.sync_copy(z_hbm.at[pl.ds(s * STRIPE, STRIPE)],
                        acc_sh.at[pl.ds(s * STRIPE, STRIPE)])
        plsc.subcore_barrier()
        pltpu.sync_copy(dst_hbm.at[wid], idx_v)
        for j in range(K):
            pltpu.sync_copy(msg_hbm.at[pl.ds((wid * K + j) * CH, CH)], row_v)
            pltpu.sync_copy(row_v, acc_sh.at[idx_v.at[j]], add=True)
        plsc.subcore_barrier()
        pltpu.sync_copy(acc_sh.at[pl.ds(s * STRIPE, STRIPE)],
                        out_hbm.at[c].at[pl.ds(s * STRIPE, STRIPE)])

    return k(msg, dst3, zeros)


def _tc_edge(ea_pad, x_src, w1p, b1r, w2, b2r, m_in, m_out):
    TILE = 256
    steps = E_PAD // TILE

    def body(ea_ref, xs_ref, w1_ref, b1_ref, w2_ref, b2_ref, out_ref):
        h = jnp.dot(ea_ref[...], w1_ref[...],
                    preferred_element_type=jnp.float32, precision=lax.Precision.HIGHEST) + b1_ref[...]
        h = jnp.maximum(h, 0.0)
        we = jnp.dot(h, w2_ref[...],
                     preferred_element_type=jnp.float32, precision=lax.Precision.HIGHEST) + b2_ref[...]
        xs = xs_ref[...]
        acc = xs[:, 0:1] * we[:, 0:m_out]
        for i in range(1, m_in):
            acc = acc + xs[:, i:i + 1] * we[:, i * m_out:(i + 1) * m_out]
        out_ref[...] = acc

    return pl.pallas_call(
        body,
        grid=(steps,),
        in_specs=[
            pl.BlockSpec((TILE, 8), lambda i: (i, 0)),
            pl.BlockSpec((TILE, m_in), lambda i: (i, 0)),
            pl.BlockSpec((8, 128), lambda i: (0, 0)),
            pl.BlockSpec((1, 128), lambda i: (0, 0)),
            pl.BlockSpec((128, m_in * m_out), lambda i: (0, 0)),
            pl.BlockSpec((1, m_in * m_out), lambda i: (0, 0)),
        ],
        out_specs=pl.BlockSpec((TILE, m_out), lambda i: (i, 0)),
        out_shape=jax.ShapeDtypeStruct((E_PAD, m_out), jnp.float32),
    )(ea_pad, x_src, w1p, b1r, w2, b2r)


def _elu(v):
    return jnp.where(v > 0, v, jnp.exp(jnp.minimum(v, 0.0)) - 1.0)


def _tc_node(parts, x, root, biasr, m_in, m_out):
    steps = N // TN

    def body(p_ref, x_ref, r_ref, b_ref, o_ref):
        v = (p_ref[0] + p_ref[1]
             + jnp.dot(x_ref[...], r_ref[...],
                       preferred_element_type=jnp.float32, precision=lax.Precision.HIGHEST)
             + b_ref[...])
        o_ref[...] = _elu(v)

    return pl.pallas_call(
        body,
        grid=(steps,),
        in_specs=[
            pl.BlockSpec((NC, TN, m_out), lambda i: (0, i, 0)),
            pl.BlockSpec((TN, m_in), lambda i: (i, 0)),
            pl.BlockSpec((m_in, m_out), lambda i: (0, 0)),
            pl.BlockSpec((1, m_out), lambda i: (0, 0)),
        ],
        out_specs=pl.BlockSpec((TN, m_out), lambda i: (i, 0)),
        out_shape=jax.ShapeDtypeStruct((N, m_out), jnp.float32),
    )(parts, x, root, biasr)


def _tc_final(parts, x, root, biasr, batch3,
              fc1_w, fc1_br, fc2_w, fc2_br, fc3_w, fc3_br):
    steps = N // TN

    def body(p_ref, x_ref, r_ref, b_ref, bat_ref,
             f1w, f1b, f2w, f2b, f3w, f3b, o_ref, acc):
        i = pl.program_id(0)

        @pl.when(i == 0)
        def _():
            acc[...] = jnp.zeros_like(acc)

        v = (p_ref[0] + p_ref[1]
             + jnp.dot(x_ref[...], r_ref[...],
                       preferred_element_type=jnp.float32, precision=lax.Precision.HIGHEST)
             + b_ref[...])
        h3 = _elu(v)
        lane = lax.broadcasted_iota(jnp.int32, (TN, 64), 1)
        ones_pad = jnp.where(lane == 0, 1.0, 0.0)
        aug = jnp.concatenate([h3, ones_pad], axis=1)
        bat = bat_ref[...].reshape(1, TN)
        g_iota = lax.broadcasted_iota(jnp.int32, (NG, TN), 0)
        oh = (g_iota == bat).astype(jnp.float32)
        acc[...] += jnp.dot(oh, aug, preferred_element_type=jnp.float32, precision=lax.Precision.HIGHEST)

        @pl.when(i == steps - 1)
        def _():
            a = acc[...]
            sums = a[:, :64]
            cnts = a[:, 64:65]
            g = sums / jnp.maximum(cnts, 1.0)
            g = _elu(jnp.dot(g, f1w[...],
                             preferred_element_type=jnp.float32, precision=lax.Precision.HIGHEST) + f1b[...])
            g = _elu(jnp.dot(g, f2w[...],
                             preferred_element_type=jnp.float32, precision=lax.Precision.HIGHEST) + f2b[...])
            g = jnp.dot(g, f3w[...],
                        preferred_element_type=jnp.float32, precision=lax.Precision.HIGHEST) + f3b[...]
            o_ref[...] = jnp.broadcast_to(g, (NG, 128))

    return pl.pallas_call(
        body,
        grid=(steps,),
        in_specs=[
            pl.BlockSpec((NC, TN, 64), lambda i: (0, i, 0)),
            pl.BlockSpec((TN, 64), lambda i: (i, 0)),
            pl.BlockSpec((64, 64), lambda i: (0, 0)),
            pl.BlockSpec((1, 64), lambda i: (0, 0)),
            pl.BlockSpec((1, 1, TN), lambda i: (i, 0, 0)),
            pl.BlockSpec((64, 32), lambda i: (0, 0)),
            pl.BlockSpec((1, 32), lambda i: (0, 0)),
            pl.BlockSpec((32, 16), lambda i: (0, 0)),
            pl.BlockSpec((1, 16), lambda i: (0, 0)),
            pl.BlockSpec((16, 1), lambda i: (0, 0)),
            pl.BlockSpec((1, 1), lambda i: (0, 0)),
        ],
        out_specs=pl.BlockSpec((NG, 128), lambda i: (0, 0)),
        out_shape=jax.ShapeDtypeStruct((NG, 128), jnp.float32),
        scratch_shapes=[pltpu.VMEM((NG, 128), jnp.float32)],
    )(parts, x, root, biasr, batch3,
      fc1_w, fc1_br, fc2_w, fc2_br, fc3_w, fc3_br)


def kernel(x, edge_index, edge_attr, batch,
           conv0_w1, conv0_b1, conv0_w2, conv0_b2, conv0_root, conv0_bias,
           conv1_w1, conv1_b1, conv1_w2, conv1_b2, conv1_root, conv1_bias,
           conv2_w1, conv2_b1, conv2_w2, conv2_b2, conv2_root, conv2_bias,
           fc1_w, fc1_b, fc2_w, fc2_b, fc3_w, fc3_b):
    src = edge_index[0].astype(jnp.int32)
    dst = edge_index[1].astype(jnp.int32)
    pad_e = E_PAD - E
    src_p = jnp.concatenate(
        [src, jnp.zeros((pad_e,), jnp.int32)]).reshape(NW, K, CH)
    dst_p = jnp.concatenate(
        [dst, jnp.full((pad_e,), N, jnp.int32)]).reshape(NW, K, CH)
    ea_pad = jnp.pad(edge_attr, ((0, pad_e), (0, 3)))
    batch3 = batch.astype(jnp.int32).reshape(N // TN, 1, TN)

    convs = [
        (conv0_w1, conv0_b1, conv0_w2, conv0_b2, conv0_root, conv0_bias),
        (conv1_w1, conv1_b1, conv1_w2, conv1_b2, conv1_root, conv1_bias),
        (conv2_w1, conv2_b1, conv2_w2, conv2_b2, conv2_root, conv2_bias),
    ]

    h = x
    for l, (m_in, m_out) in enumerate(DIMS):
        w1, b1, w2, b2, root, bias = convs[l]
        w1p = jnp.pad(w1, ((0, 3), (0, 0)))
        b1r = b1.reshape(1, 128)
        b2r = b2.reshape(1, m_in * m_out)
        biasr = bias.reshape(1, m_out)
        zeros = jnp.zeros((N_PAD, m_out), jnp.float32)

        xs = _sc_gather(h, src_p, m_in)
        msg = _tc_edge(ea_pad, xs, w1p, b1r, w2, b2r, m_in, m_out)
        parts = _sc_scatter(msg, dst_p, zeros, m_out)
        if l < 2:
            h = _tc_node(parts, h, root, biasr, m_in, m_out)
        else:
            res = _tc_final(parts, h, root, biasr, batch3,
                            fc1_w, fc1_b.reshape(1, 32),
                            fc2_w, fc2_b.reshape(1, 16),
                            fc3_w, fc3_b.reshape(1, 1))
    return res[:, 0]
